# Initial kernel scaffold; baseline (speedup 1.0000x reference)
import jax
import jax.numpy as jnp
from jax.experimental import pallas as pl


def kernel(features, edge_index):
    raise NotImplementedError("write your pallas kernel here")


# speedup vs baseline: 4.9525x; 4.9525x over previous
import functools

import jax
import jax.numpy as jnp
from jax import lax
from jax.experimental import pallas as pl
from jax.experimental.pallas import tpu as pltpu
from jax.experimental.pallas import tpu_sc as plsc

N_NODES = 10000
N_EDGES = 320000
D = 128
EPS = 1e-8

NC, NS = 2, 16
NW = NC * NS
B = 128
NB = 80
NSTAGE = 5
NBC = NB // NSTAGE
EPW = NB * B
EP = NW * EPW
ROWS_PER_TILE = 632
ACC_ROWS = NS * ROWS_PER_TILE
DUMMY = N_NODES


def _norm_body(f_ref, o_ref):
    x = f_ref[...]
    n2 = jnp.sum(x * x, axis=1, keepdims=True)
    inv = 1.0 / jnp.maximum(jnp.sqrt(n2), EPS)
    o_ref[...] = x * inv


def _sc_body(fn_hbm, src_hbm, dst_hbm, zr_hbm, zc_hbm, g_hbm, cnt_hbm,
             acc, s_all, m_all, cnt_l, r0, r1, semz, sem0, sem1):
    c = lax.axis_index("c")
    s = lax.axis_index("s")
    wid = s * NC + c
    row_base = s * ROWS_PER_TILE

    hz = pltpu.async_copy(zr_hbm, acc.at[pl.ds(row_base, ROWS_PER_TILE)], semz)
    pltpu.sync_copy(zc_hbm, cnt_l)
    hz.wait()
    plsc.subcore_barrier()

    ones16 = jnp.ones((16,), jnp.float32)

    for st in range(NSTAGE):
        pltpu.sync_copy(src_hbm.at[wid, pl.ds(st * NBC, NBC)], s_all)
        pltpu.sync_copy(dst_hbm.at[wid, pl.ds(st * NBC, NBC)], m_all)

        def mask_row(r, carry):
            for t in range(B // 16):
                sl = pl.ds(t * 16, 16)
                sv = s_all[r, sl]
                dv = m_all[r, sl]
                dm = jnp.where(sv == dv, DUMMY, dv)
                m_all[r, sl] = dm
                plsc.addupdate_scatter(cnt_l, [dm], ones16)
            return carry

        lax.fori_loop(0, NBC, mask_row, 0)

        pltpu.async_copy(fn_hbm.at[s_all.at[0]], r0, sem0)

        def body(k, carry):
            j = 2 * k
            pltpu.async_copy(fn_hbm.at[s_all.at[j + 1]], r1, sem1)
            pltpu.make_async_copy(fn_hbm.at[s_all.at[j]], r0, sem0).wait()
            pltpu.sync_copy(r0, acc.at[m_all.at[j]], add=True)

            @pl.when(k < NBC // 2 - 1)
            def _():
                pltpu.async_copy(fn_hbm.at[s_all.at[j + 2]], r0, sem0)

            pltpu.make_async_copy(fn_hbm.at[s_all.at[j + 1]], r1, sem1).wait()
            pltpu.sync_copy(r1, acc.at[m_all.at[j + 1]], add=True)
            return carry

        lax.fori_loop(0, NBC // 2, body, 0)

    plsc.subcore_barrier()
    out_off = c * ACC_ROWS + row_base
    pltpu.sync_copy(acc.at[pl.ds(row_base, ROWS_PER_TILE)],
                    g_hbm.at[pl.ds(out_off, ROWS_PER_TILE)])
    pltpu.sync_copy(cnt_l, cnt_hbm.at[wid])


_sc_scatter = functools.partial(
    pl.kernel,
    mesh=plsc.VectorSubcoreMesh(core_axis_name="c", subcore_axis_name="s"),
    compiler_params=pltpu.CompilerParams(needs_layout_passes=False),
    out_type=(
        jax.ShapeDtypeStruct((NC * ACC_ROWS, D), jnp.float32),
        jax.ShapeDtypeStruct((NW, ACC_ROWS), jnp.float32),
    ),
    scratch_types=[
        pltpu.VMEM_SHARED((ACC_ROWS, D), jnp.float32),
        pltpu.VMEM((NBC, B), jnp.int32),
        pltpu.VMEM((NBC, B), jnp.int32),
        pltpu.VMEM((ACC_ROWS,), jnp.float32),
        pltpu.VMEM((B, D), jnp.float32),
        pltpu.VMEM((B, D), jnp.float32),
        pltpu.SemaphoreType.DMA,
        pltpu.SemaphoreType.DMA,
        pltpu.SemaphoreType.DMA,
    ],
)(_sc_body)


def _final_body(f_ref, g_ref, c_ref, o_ref):
    gsum = g_ref[0:N_NODES, :] + g_ref[ACC_ROWS:ACC_ROWS + N_NODES, :]
    tot = jnp.sum(f_ref[...] * gsum, axis=1)
    cnt = jnp.sum(c_ref[...], axis=0)[0:N_NODES]
    o_ref[...] = jnp.where(cnt > 0, tot / jnp.maximum(cnt, 1.0), 0.0)


def kernel(features, edge_index):
    fn = pl.pallas_call(
        _norm_body,
        out_shape=jax.ShapeDtypeStruct((N_NODES, D), jnp.float32),
    )(features)

    pad = EP - N_EDGES
    zpad = jnp.zeros((pad,), jnp.int32)
    srcp = jnp.concatenate([edge_index[0], zpad]).reshape(NW, NB, B)
    dstp = jnp.concatenate([edge_index[1], zpad]).reshape(NW, NB, B)
    zrows = jnp.zeros((ROWS_PER_TILE, D), jnp.float32)
    zcnt = jnp.zeros((ACC_ROWS,), jnp.float32)

    g, cnt = _sc_scatter(fn, srcp, dstp, zrows, zcnt)

    out = pl.pallas_call(
        _final_body,
        out_shape=jax.ShapeDtypeStruct((N_NODES,), jnp.float32),
    )(fn, g, cnt)
    return out
